# initial kernel scaffold (unmeasured)
import jax
import jax.numpy as jnp
from jax import lax
from jax.experimental import pallas as pl
from jax.experimental.pallas import tpu as pltpu

N_DEV = 32
B, D = 512, 256
CH = B // N_DEV
N_LAYER = 3


def kernel(x, Win0, Wout0, Win1, Wout1, Win2, Wout2):
    def body(
        x_ref,
        win0_ref,
        wout0_ref,
        win1_ref,
        wout1_ref,
        win2_ref,
        wout2_ref,
        out_ref,
        send_buf,
        red_buf,
        rs_buf,
        ag_buf,
        rs_sems,
        ag_sems,
        send_sems,
    ):
        me = lax.axis_index("i")

        bar = pltpu.get_barrier_semaphore()
        for k in range(1, N_DEV):
            pl.semaphore_signal(
                bar,
                inc=1,
                device_id=((me + k) % N_DEV,),
                device_id_type=pl.DeviceIdType.MESH,
            )
        pl.semaphore_wait(bar, N_DEV - 1)

        wins = [win0_ref, win1_ref, win2_ref]
        wouts = [wout0_ref, wout1_ref, wout2_ref]

        xv = x_ref[:, :].astype(jnp.bfloat16)
        for l in range(N_LAYER):
            w_in = wins[l][:, :].astype(jnp.bfloat16)
            w_out = wouts[l][:, :].astype(jnp.bfloat16)
            h = jnp.dot(xv, w_in, preferred_element_type=jnp.float32)
            h = jnp.maximum(h, 0.0).astype(jnp.bfloat16)
            p = jnp.dot(h, w_out, preferred_element_type=jnp.float32)
            send_buf[:, :] = p.astype(jnp.bfloat16)

            sends = []
            for k in range(1, N_DEV):
                d = (me + k) % N_DEV
                rdma = pltpu.make_async_remote_copy(
                    src_ref=send_buf.at[pl.ds(d * CH, CH), :],
                    dst_ref=rs_buf.at[l, N_DEV - k],
                    send_sem=send_sems.at[k],
                    recv_sem=rs_sems.at[l, N_DEV - k],
                    device_id=(d,),
                    device_id_type=pl.DeviceIdType.MESH,
                )
                rdma.start()
                sends.append(rdma)
            rs_buf[l, 0] = lax.dynamic_slice(p, (me * CH, 0), (CH, D)).astype(
                jnp.bfloat16
            )
            for j in range(1, N_DEV):
                pltpu.make_async_remote_copy(
                    src_ref=rs_buf.at[l, j],
                    dst_ref=rs_buf.at[l, j],
                    send_sem=send_sems.at[0],
                    recv_sem=rs_sems.at[l, j],
                    device_id=(0,),
                    device_id_type=pl.DeviceIdType.MESH,
                ).wait_recv()
            for r in sends:
                r.wait_send()

            red = jnp.sum(rs_buf[l].astype(jnp.float32), axis=0)
            red_buf[l] = red.astype(jnp.bfloat16)

            sends = []
            for k in range(1, N_DEV):
                d = (me + k) % N_DEV
                rdma = pltpu.make_async_remote_copy(
                    src_ref=red_buf.at[l],
                    dst_ref=ag_buf.at[l, N_DEV - k],
                    send_sem=send_sems.at[k],
                    recv_sem=ag_sems.at[l, N_DEV - k],
                    device_id=(d,),
                    device_id_type=pl.DeviceIdType.MESH,
                )
                rdma.start()
                sends.append(rdma)
            ag_buf[l, 0] = red_buf[l]
            for j in range(1, N_DEV):
                pltpu.make_async_remote_copy(
                    src_ref=ag_buf.at[l, j],
                    dst_ref=ag_buf.at[l, j],
                    send_sem=send_sems.at[0],
                    recv_sem=ag_sems.at[l, j],
                    device_id=(0,),
                    device_id_type=pl.DeviceIdType.MESH,
                ).wait_recv()
            for r in sends:
                r.wait_send()

            rel = ag_buf[l]
            ordered = jnp.roll(rel, me, axis=0)
            xv = ordered.reshape(B, D)

        out_ref[:, :] = xv.astype(jnp.float32)

    return pl.pallas_call(
        body,
        out_shape=jax.ShapeDtypeStruct((B, D), jnp.float32),
        in_specs=[pl.BlockSpec(memory_space=pltpu.VMEM)] * 7,
        out_specs=pl.BlockSpec(memory_space=pltpu.VMEM),
        scratch_shapes=[
            pltpu.VMEM((B, D), jnp.bfloat16),
            pltpu.VMEM((N_LAYER, CH, D), jnp.bfloat16),
            pltpu.VMEM((N_LAYER, N_DEV, CH, D), jnp.bfloat16),
            pltpu.VMEM((N_LAYER, N_DEV, CH, D), jnp.bfloat16),
            pltpu.SemaphoreType.DMA((N_LAYER, N_DEV)),
            pltpu.SemaphoreType.DMA((N_LAYER, N_DEV)),
            pltpu.SemaphoreType.DMA((N_DEV,)),
        ],
        compiler_params=pltpu.CompilerParams(collective_id=0),
    )(x, Win0, Wout0, Win1, Wout1, Win2, Wout2)


# baseline (device time: 54752 ns/iter reference)
import jax
import jax.numpy as jnp
from jax import lax
from jax.experimental import pallas as pl
from jax.experimental.pallas import tpu as pltpu

N_DEV = 32
B, D = 512, 256
CH = B // N_DEV
N_LAYER = 3


def kernel(x, Win0, Wout0, Win1, Wout1, Win2, Wout2):
    def body(
        x_ref,
        win0_ref,
        wout0_ref,
        win1_ref,
        wout1_ref,
        win2_ref,
        wout2_ref,
        out_ref,
        send_buf,
        red_buf,
        rs_buf,
        ag_buf,
        rs_sems,
        ag_sems,
        send_sems,
    ):
        me = lax.axis_index("i")

        bar = pltpu.get_barrier_semaphore()
        for k in range(1, N_DEV):
            pl.semaphore_signal(
                bar,
                inc=1,
                device_id=((me + k) % N_DEV,),
                device_id_type=pl.DeviceIdType.MESH,
            )
        pl.semaphore_wait(bar, N_DEV - 1)

        wins = [win0_ref, win1_ref, win2_ref]
        wouts = [wout0_ref, wout1_ref, wout2_ref]

        xv = x_ref[:, :].astype(jnp.bfloat16)
        for l in range(N_LAYER):
            w_in = wins[l][:, :].astype(jnp.bfloat16)
            w_out = wouts[l][:, :].astype(jnp.bfloat16)
            h = jnp.dot(xv, w_in, preferred_element_type=jnp.float32)
            h = jnp.maximum(h, 0.0).astype(jnp.bfloat16)
            p = jnp.dot(h, w_out, preferred_element_type=jnp.float32)
            send_buf[:, :] = p.astype(jnp.bfloat16)

            sends = []
            for k in range(1, N_DEV):
                d = (me + k) % N_DEV
                rdma = pltpu.make_async_remote_copy(
                    src_ref=send_buf.at[pl.ds(d * CH, CH), :],
                    dst_ref=rs_buf.at[l, N_DEV - k],
                    send_sem=send_sems.at[k],
                    recv_sem=rs_sems.at[l, N_DEV - k],
                    device_id=(d,),
                    device_id_type=pl.DeviceIdType.MESH,
                )
                rdma.start()
                sends.append(rdma)
            rs_buf[l, 0] = send_buf[pl.ds(me * CH, CH), :]
            for j in range(1, N_DEV):
                pltpu.make_async_remote_copy(
                    src_ref=rs_buf.at[l, j],
                    dst_ref=rs_buf.at[l, j],
                    send_sem=send_sems.at[0],
                    recv_sem=rs_sems.at[l, j],
                    device_id=(0,),
                    device_id_type=pl.DeviceIdType.MESH,
                ).wait_recv()
            for r in sends:
                r.wait_send()

            red = jnp.sum(rs_buf[l].astype(jnp.float32), axis=0)
            red_buf[l] = red.astype(jnp.bfloat16)

            sends = []
            for k in range(1, N_DEV):
                d = (me + k) % N_DEV
                rdma = pltpu.make_async_remote_copy(
                    src_ref=red_buf.at[l],
                    dst_ref=ag_buf.at[l].at[pl.ds(me * CH, CH), :],
                    send_sem=send_sems.at[k],
                    recv_sem=ag_sems.at[l, N_DEV - k],
                    device_id=(d,),
                    device_id_type=pl.DeviceIdType.MESH,
                )
                rdma.start()
                sends.append(rdma)
            ag_buf[l, pl.ds(me * CH, CH), :] = red_buf[l]
            for j in range(1, N_DEV):
                pltpu.make_async_remote_copy(
                    src_ref=ag_buf.at[l].at[pl.ds(0, CH), :],
                    dst_ref=ag_buf.at[l].at[pl.ds(0, CH), :],
                    send_sem=send_sems.at[0],
                    recv_sem=ag_sems.at[l, j],
                    device_id=(0,),
                    device_id_type=pl.DeviceIdType.MESH,
                ).wait_recv()
            for r in sends:
                r.wait_send()

            xv = ag_buf[l]

        out_ref[:, :] = xv.astype(jnp.float32)

    return pl.pallas_call(
        body,
        out_shape=jax.ShapeDtypeStruct((B, D), jnp.float32),
        in_specs=[pl.BlockSpec(memory_space=pltpu.VMEM)] * 7,
        out_specs=pl.BlockSpec(memory_space=pltpu.VMEM),
        scratch_shapes=[
            pltpu.VMEM((B, D), jnp.bfloat16),
            pltpu.VMEM((N_LAYER, CH, D), jnp.bfloat16),
            pltpu.VMEM((N_LAYER, N_DEV, CH, D), jnp.bfloat16),
            pltpu.VMEM((N_LAYER, B, D), jnp.bfloat16),
            pltpu.SemaphoreType.DMA((N_LAYER, N_DEV)),
            pltpu.SemaphoreType.DMA((N_LAYER, N_DEV)),
            pltpu.SemaphoreType.DMA((N_DEV,)),
        ],
        compiler_params=pltpu.CompilerParams(collective_id=0),
    )(x, Win0, Wout0, Win1, Wout1, Win2, Wout2)


# device time: 54080 ns/iter; 1.0124x vs baseline; 1.0124x over previous
import jax
import jax.numpy as jnp
from jax import lax
from jax.experimental import pallas as pl
from jax.experimental.pallas import tpu as pltpu

N_DEV = 32
B, D = 512, 256
CH = B // N_DEV
N_LAYER = 3


def kernel(x, Win0, Wout0, Win1, Wout1, Win2, Wout2):
    def body(
        x_ref,
        win0_ref,
        wout0_ref,
        win1_ref,
        wout1_ref,
        win2_ref,
        wout2_ref,
        out_ref,
        send_buf,
        red_buf,
        rs_buf,
        ag_buf,
        rs_sems,
        ag_sems,
        send_sems,
    ):
        me = lax.axis_index("i")

        bar = pltpu.get_barrier_semaphore()
        for k in range(1, N_DEV):
            pl.semaphore_signal(
                bar,
                inc=1,
                device_id=((me + k) % N_DEV,),
                device_id_type=pl.DeviceIdType.MESH,
            )

        wins = [win0_ref, win1_ref, win2_ref]
        wouts = [wout0_ref, wout1_ref, wout2_ref]

        def fwd(l, xv):
            w_in = wins[l][:, :].astype(jnp.bfloat16)
            w_out = wouts[l][:, :].astype(jnp.bfloat16)
            h = jnp.dot(xv, w_in, preferred_element_type=jnp.float32)
            h = jnp.maximum(h, 0.0).astype(jnp.bfloat16)
            return jnp.dot(h, w_out, preferred_element_type=jnp.float32)

        xv = x_ref[:, :].astype(jnp.bfloat16)
        p = fwd(0, xv)
        send_buf[:, :] = p.astype(jnp.bfloat16)
        pl.semaphore_wait(bar, N_DEV - 1)

        for l in range(N_LAYER):
            if l > 0:
                p = fwd(l, xv)
                send_buf[:, :] = p.astype(jnp.bfloat16)

            sends = []
            for k in range(1, N_DEV):
                d = (me + k) % N_DEV
                rdma = pltpu.make_async_remote_copy(
                    src_ref=send_buf.at[pl.ds(d * CH, CH), :],
                    dst_ref=rs_buf.at[l, N_DEV - k],
                    send_sem=send_sems.at[k],
                    recv_sem=rs_sems.at[l, N_DEV - k],
                    device_id=(d,),
                    device_id_type=pl.DeviceIdType.MESH,
                )
                rdma.start()
                sends.append(rdma)
            rs_buf[l, 0] = send_buf[pl.ds(me * CH, CH), :]
            for j in range(1, N_DEV):
                pltpu.make_async_remote_copy(
                    src_ref=rs_buf.at[l, j],
                    dst_ref=rs_buf.at[l, j],
                    send_sem=send_sems.at[0],
                    recv_sem=rs_sems.at[l, j],
                    device_id=(0,),
                    device_id_type=pl.DeviceIdType.MESH,
                ).wait_recv()
            for r in sends:
                r.wait_send()

            red = jnp.sum(rs_buf[l].astype(jnp.float32), axis=0)
            red_buf[l] = red.astype(jnp.bfloat16)

            sends = []
            for k in range(1, N_DEV):
                d = (me + k) % N_DEV
                rdma = pltpu.make_async_remote_copy(
                    src_ref=red_buf.at[l],
                    dst_ref=ag_buf.at[l].at[pl.ds(me * CH, CH), :],
                    send_sem=send_sems.at[k],
                    recv_sem=ag_sems.at[l, N_DEV - k],
                    device_id=(d,),
                    device_id_type=pl.DeviceIdType.MESH,
                )
                rdma.start()
                sends.append(rdma)
            ag_buf[l, pl.ds(me * CH, CH), :] = red_buf[l]
            for j in range(1, N_DEV):
                pltpu.make_async_remote_copy(
                    src_ref=ag_buf.at[l].at[pl.ds(0, CH), :],
                    dst_ref=ag_buf.at[l].at[pl.ds(0, CH), :],
                    send_sem=send_sems.at[0],
                    recv_sem=ag_sems.at[l, j],
                    device_id=(0,),
                    device_id_type=pl.DeviceIdType.MESH,
                ).wait_recv()
            for r in sends:
                r.wait_send()

            xv = ag_buf[l]

        out_ref[:, :] = xv.astype(jnp.float32)

    return pl.pallas_call(
        body,
        out_shape=jax.ShapeDtypeStruct((B, D), jnp.float32),
        in_specs=[pl.BlockSpec(memory_space=pltpu.VMEM)] * 7,
        out_specs=pl.BlockSpec(memory_space=pltpu.VMEM),
        scratch_shapes=[
            pltpu.VMEM((B, D), jnp.bfloat16),
            pltpu.VMEM((N_LAYER, CH, D), jnp.bfloat16),
            pltpu.VMEM((N_LAYER, N_DEV, CH, D), jnp.bfloat16),
            pltpu.VMEM((N_LAYER, B, D), jnp.bfloat16),
            pltpu.SemaphoreType.DMA((N_LAYER, N_DEV)),
            pltpu.SemaphoreType.DMA((N_LAYER, N_DEV)),
            pltpu.SemaphoreType.DMA((N_DEV,)),
        ],
        compiler_params=pltpu.CompilerParams(collective_id=0),
    )(x, Win0, Wout0, Win1, Wout1, Win2, Wout2)
